# baseline (speedup 1.0000x reference)
import functools

import jax
import jax.numpy as jnp
from jax import lax
from jax.experimental import pallas as pl
from jax.experimental.pallas import tpu as pltpu
from jax.experimental.pallas import tpu_sc as plsc

KNN_K = 16
KNN_D = 2
NUM_EXTRACT = KNN_K * KNN_D - 1
B = 8
N = 2048
C = 64
ROWS = 256

NUM_WORKERS = 32
PTS_PER_W = (B * N) // NUM_WORKERS
OUT_CH = 10
CHUNK_PTS = 128
CHUNK_F32 = CHUNK_PTS * KNN_K * OUT_CH


NQ = 4
GW = N // NQ
NCAND = NQ * NUM_EXTRACT


def _topk_body(xb_ref, xf_ref, idx_ref, dscr, wscr):
    xb = xb_ref[0]
    xf = xf_ref[0]
    sqb = jnp.sum(xb * xb, axis=1, keepdims=True)
    g = lax.dot_general(xb.astype(jnp.bfloat16), xf.astype(jnp.bfloat16),
                        (((1,), (1,)), ((), ())),
                        preferred_element_type=jnp.float32)
    sqf_row = lax.dot_general(jnp.ones((1, C), jnp.float32), xf * xf,
                              (((1,), (1,)), ((), ())),
                              preferred_element_type=jnp.float32,
                              precision=lax.Precision.HIGHEST)
    d = (sqb + (-2.0 * g)) + sqf_row
    dscr[...] = d
    w = d[:, :GW]
    for qi in range(1, NQ):
        dq = d[:, qi * GW:(qi + 1) * GW]
        w = jnp.where(dq < w, dq, w)
    wscr[...] = w

    fcol = lax.broadcasted_iota(jnp.int32, (ROWS, GW), 1)
    lane128 = lax.broadcasted_iota(jnp.int32, (ROWS, 128), 1)
    lane16 = lax.broadcasted_iota(jnp.int32, (ROWS, KNN_K), 1)
    inf = jnp.float32(jnp.inf)

    def pop_group(j, cand):
        candv, candc = cand
        wc = wscr[...]
        m = jnp.min(wc, axis=1, keepdims=True)
        am = jnp.min(jnp.where(wc == m, fcol, GW), axis=1, keepdims=True)
        msk = fcol == am
        wscr[...] = jnp.where(msk, inf, wc)
        for qi in range(NQ):
            dq = dscr[:, qi * GW:(qi + 1) * GW]
            lv = jnp.min(jnp.where(msk, dq, inf), axis=1, keepdims=True)
            tgt = lane128 == (NQ * j + qi)
            candv = jnp.where(tgt, lv, candv)
            candc = jnp.where(tgt, am + qi * GW, candc)
        return candv, candc

    candv, candc = lax.fori_loop(
        0, NUM_EXTRACT, pop_group,
        (jnp.full((ROWS, 128), inf, jnp.float32),
         jnp.full((ROWS, 128), N, jnp.int32)))

    def pop_final(j, state):
        cv, cc, acc = state
        m = jnp.min(cv, axis=1, keepdims=True)
        ac = jnp.min(jnp.where(cv == m, cc, N), axis=1, keepdims=True)
        cv = jnp.where((cv == m) & (cc == ac), inf, cv)
        keep = (j % 2 == 0) & (lane16 == (j // 2))
        return cv, cc, jnp.where(keep, ac, acc)

    _, _, acc = lax.fori_loop(
        0, NUM_EXTRACT, pop_final,
        (candv, candc, jnp.zeros((ROWS, KNN_K), jnp.int32)))
    idx_ref[0] = acc


def _dilated_knn_idx(new_points):
    return pl.pallas_call(
        _topk_body,
        grid=(B, N // ROWS),
        in_specs=[
            pl.BlockSpec((1, ROWS, C), lambda b, r: (b, r, 0)),
            pl.BlockSpec((1, N, C), lambda b, r: (b, 0, 0)),
        ],
        out_specs=pl.BlockSpec((1, ROWS, KNN_K), lambda b, r: (b, r, 0)),
        out_shape=jax.ShapeDtypeStruct((B, N, KNN_K), jnp.int32),
        scratch_shapes=[pltpu.VMEM((ROWS, N), jnp.float32),
                        pltpu.VMEM((ROWS, GW), jnp.float32)],
    )(new_points, new_points)


def _sqrt16(d2):
    bits = lax.bitcast_convert_type(d2, jnp.int32)
    y = lax.bitcast_convert_type(
        jnp.int32(0x5F3759DF) - (bits >> 1), jnp.float32)
    for _ in range(3):
        y = y * (1.5 - 0.5 * d2 * y * y)
    return jnp.where(d2 > 0.0, d2 * y, 0.0)


def _encode_body(xyz_ref, nn_ref, out_ref, tv, iv, ov):
    cid = lax.axis_index("c")
    sid = lax.axis_index("s")
    w = sid * 2 + cid
    pltpu.sync_copy(xyz_ref, tv)
    pltpu.sync_copy(nn_ref.at[pl.ds(w * (PTS_PER_W * KNN_K), PTS_PER_W * KNN_K)], iv)
    base = (w // (N // PTS_PER_W)) * N
    lanes = jnp.arange(16, dtype=jnp.int32)

    for sub in range(PTS_PER_W // CHUNK_PTS):
        def body(p2, carry):
            p_loc = sub * CHUNK_PTS + p2
            idx16 = iv[pl.ds(p_loc * KNN_K, KNN_K)]
            gi = (idx16 + base) * 3
            nx = plsc.load_gather(tv, [gi])
            ny = plsc.load_gather(tv, [gi + 1])
            nz = plsc.load_gather(tv, [gi + 2])
            pw = (w * PTS_PER_W + p_loc) * 3
            own_i = jnp.full((16,), pw, jnp.int32)
            ox = plsc.load_gather(tv, [own_i])
            oy = plsc.load_gather(tv, [own_i + 1])
            oz = plsc.load_gather(tv, [own_i + 2])
            rx = ox - nx
            ry = oy - ny
            rz = oz - nz
            dist = _sqrt16(rx * rx + ry * ry + rz * rz)
            offs = p2 * (KNN_K * OUT_CH) + lanes * OUT_CH
            plsc.store_scatter(ov, [offs], dist)
            plsc.store_scatter(ov, [offs + 1], rx)
            plsc.store_scatter(ov, [offs + 2], ry)
            plsc.store_scatter(ov, [offs + 3], rz)
            plsc.store_scatter(ov, [offs + 4], ox)
            plsc.store_scatter(ov, [offs + 5], oy)
            plsc.store_scatter(ov, [offs + 6], oz)
            plsc.store_scatter(ov, [offs + 7], nx)
            plsc.store_scatter(ov, [offs + 8], ny)
            plsc.store_scatter(ov, [offs + 9], nz)
            return carry

        lax.fori_loop(0, CHUNK_PTS, body, 0)
        pltpu.sync_copy(
            ov, out_ref.at[pl.ds(w * (PTS_PER_W * KNN_K * OUT_CH)
                                 + sub * CHUNK_F32, CHUNK_F32)])


def _relative_pos_encode(xyz_flat, nn_flat):
    mesh = plsc.VectorSubcoreMesh(core_axis_name="c", subcore_axis_name="s")
    run = functools.partial(
        pl.kernel,
        mesh=mesh,
        compiler_params=pltpu.CompilerParams(needs_layout_passes=False),
        out_type=jax.ShapeDtypeStruct((B * N * KNN_K * OUT_CH,), jnp.float32),
        scratch_types=[
            pltpu.VMEM((B * N * 3,), jnp.float32),
            pltpu.VMEM((PTS_PER_W * KNN_K,), jnp.int32),
            pltpu.VMEM((CHUNK_F32,), jnp.float32),
        ],
    )(_encode_body)
    return run(xyz_flat, nn_flat)


def kernel(new_xyz, new_points):
    nn_idx = _dilated_knn_idx(new_points)
    out = _relative_pos_encode(new_xyz.reshape(-1), nn_idx.reshape(-1))
    return out.reshape(B, N, KNN_K, OUT_CH)


# speedup vs baseline: 1.4591x; 1.4591x over previous
import functools

import jax
import jax.numpy as jnp
from jax import lax
from jax.experimental import pallas as pl
from jax.experimental.pallas import tpu as pltpu
from jax.experimental.pallas import tpu_sc as plsc

KNN_K = 16
KNN_D = 2
NUM_EXTRACT = KNN_K * KNN_D - 1
B = 8
N = 2048
C = 64
ROWS = 512

NUM_WORKERS = 32
PTS_PER_W = (B * N) // NUM_WORKERS
OUT_CH = 10
CHUNK_PTS = 128
CHUNK_F32 = CHUNK_PTS * KNN_K * OUT_CH


def _topk_body(xb_ref, xf_ref, idx_ref, dscr, xfbscr, sqfscr):
    xb = xb_ref[0]

    @pl.when(pl.program_id(1) == 0)
    def _():
        xf = xf_ref[0]
        xfbscr[...] = xf.astype(jnp.bfloat16)
        sqfscr[...] = lax.dot_general(jnp.ones((1, C), jnp.float32), xf * xf,
                                      (((1,), (1,)), ((), ())),
                                      preferred_element_type=jnp.float32,
                                      precision=lax.Precision.HIGHEST)

    sqb = jnp.sum(xb * xb, axis=1, keepdims=True)
    g = lax.dot_general(xb.astype(jnp.bfloat16), xfbscr[...],
                        (((1,), (1,)), ((), ())),
                        preferred_element_type=jnp.float32)
    dscr[...] = (sqb + (-2.0 * g)) + sqfscr[...]

    colid = lax.broadcasted_iota(jnp.int32, (ROWS, N), 1)
    lane16 = lax.broadcasted_iota(jnp.int32, (ROWS, KNN_K), 1)
    inf = jnp.float32(jnp.inf)

    def step(j, acc):
        dc = dscr[...]
        m = jnp.min(dc, axis=1, keepdims=True)
        am = jnp.min(jnp.where(dc == m, colid, N), axis=1, keepdims=True)
        dscr[...] = jnp.where(colid == am, inf, dc)
        keep = (j % 2 == 0) & (lane16 == (j // 2))
        return jnp.where(keep, am, acc)

    acc = lax.fori_loop(0, NUM_EXTRACT, step,
                        jnp.zeros((ROWS, KNN_K), jnp.int32))
    idx_ref[0] = acc


def _dilated_knn_idx(new_points):
    return pl.pallas_call(
        _topk_body,
        grid=(B, N // ROWS),
        in_specs=[
            pl.BlockSpec((1, ROWS, C), lambda b, r: (b, r, 0)),
            pl.BlockSpec((1, N, C), lambda b, r: (b, 0, 0)),
        ],
        out_specs=pl.BlockSpec((1, ROWS, KNN_K), lambda b, r: (b, r, 0)),
        out_shape=jax.ShapeDtypeStruct((B, N, KNN_K), jnp.int32),
        scratch_shapes=[pltpu.VMEM((ROWS, N), jnp.float32),
                        pltpu.VMEM((N, C), jnp.bfloat16),
                        pltpu.VMEM((1, N), jnp.float32)],
    )(new_points, new_points)


def _sqrt16(d2):
    bits = lax.bitcast_convert_type(d2, jnp.int32)
    y = lax.bitcast_convert_type(
        jnp.int32(0x5F3759DF) - (bits >> 1), jnp.float32)
    for _ in range(3):
        y = y * (1.5 - 0.5 * d2 * y * y)
    return jnp.where(d2 > 0.0, d2 * y, 0.0)


def _encode_body(xyz_ref, nn_ref, out_ref, tv, iv, ov):
    cid = lax.axis_index("c")
    sid = lax.axis_index("s")
    w = sid * 2 + cid
    pltpu.sync_copy(xyz_ref, tv)
    pltpu.sync_copy(nn_ref.at[pl.ds(w * (PTS_PER_W * KNN_K), PTS_PER_W * KNN_K)], iv)
    base = (w // (N // PTS_PER_W)) * N
    lanes = jnp.arange(16, dtype=jnp.int32)

    for sub in range(PTS_PER_W // CHUNK_PTS):
        def body(p2, carry):
            p_loc = sub * CHUNK_PTS + p2
            idx16 = iv[pl.ds(p_loc * KNN_K, KNN_K)]
            gi = (idx16 + base) * 3
            nx = plsc.load_gather(tv, [gi])
            ny = plsc.load_gather(tv, [gi + 1])
            nz = plsc.load_gather(tv, [gi + 2])
            pw = (w * PTS_PER_W + p_loc) * 3
            own_i = jnp.full((16,), pw, jnp.int32)
            ox = plsc.load_gather(tv, [own_i])
            oy = plsc.load_gather(tv, [own_i + 1])
            oz = plsc.load_gather(tv, [own_i + 2])
            rx = ox - nx
            ry = oy - ny
            rz = oz - nz
            dist = _sqrt16(rx * rx + ry * ry + rz * rz)
            offs = p2 * (KNN_K * OUT_CH) + lanes * OUT_CH
            plsc.store_scatter(ov, [offs], dist)
            plsc.store_scatter(ov, [offs + 1], rx)
            plsc.store_scatter(ov, [offs + 2], ry)
            plsc.store_scatter(ov, [offs + 3], rz)
            plsc.store_scatter(ov, [offs + 4], ox)
            plsc.store_scatter(ov, [offs + 5], oy)
            plsc.store_scatter(ov, [offs + 6], oz)
            plsc.store_scatter(ov, [offs + 7], nx)
            plsc.store_scatter(ov, [offs + 8], ny)
            plsc.store_scatter(ov, [offs + 9], nz)
            return carry

        lax.fori_loop(0, CHUNK_PTS, body, 0)
        pltpu.sync_copy(
            ov, out_ref.at[pl.ds(w * (PTS_PER_W * KNN_K * OUT_CH)
                                 + sub * CHUNK_F32, CHUNK_F32)])


def _relative_pos_encode(xyz_flat, nn_flat):
    mesh = plsc.VectorSubcoreMesh(core_axis_name="c", subcore_axis_name="s")
    run = functools.partial(
        pl.kernel,
        mesh=mesh,
        compiler_params=pltpu.CompilerParams(needs_layout_passes=False),
        out_type=jax.ShapeDtypeStruct((B * N * KNN_K * OUT_CH,), jnp.float32),
        scratch_types=[
            pltpu.VMEM((B * N * 3,), jnp.float32),
            pltpu.VMEM((PTS_PER_W * KNN_K,), jnp.int32),
            pltpu.VMEM((CHUNK_F32,), jnp.float32),
        ],
    )(_encode_body)
    return run(xyz_flat, nn_flat)


def kernel(new_xyz, new_points):
    nn_idx = _dilated_knn_idx(new_points)
    out = _relative_pos_encode(new_xyz.reshape(-1), nn_idx.reshape(-1))
    return out.reshape(B, N, KNN_K, OUT_CH)


# speedup vs baseline: 1.4808x; 1.0149x over previous
import functools

import jax
import jax.numpy as jnp
from jax import lax
from jax.experimental import pallas as pl
from jax.experimental.pallas import tpu as pltpu
from jax.experimental.pallas import tpu_sc as plsc

KNN_K = 16
KNN_D = 2
NUM_EXTRACT = KNN_K * KNN_D - 1
B = 8
N = 2048
C = 64
ROWS = 1024

NUM_WORKERS = 32
PTS_PER_W = (B * N) // NUM_WORKERS
OUT_CH = 10
CHUNK_PTS = 128
CHUNK_F32 = CHUNK_PTS * KNN_K * OUT_CH


def _topk_body(xb_ref, xf_ref, idx_ref, dscr, xfbscr, sqfscr):
    xb = xb_ref[0]

    @pl.when(pl.program_id(1) == 0)
    def _():
        xf = xf_ref[0]
        xfbscr[...] = xf.astype(jnp.bfloat16)
        sqfscr[...] = lax.dot_general(jnp.ones((1, C), jnp.float32), xf * xf,
                                      (((1,), (1,)), ((), ())),
                                      preferred_element_type=jnp.float32,
                                      precision=lax.Precision.HIGHEST)

    sqb = jnp.sum(xb * xb, axis=1, keepdims=True)
    g = lax.dot_general(xb.astype(jnp.bfloat16), xfbscr[...],
                        (((1,), (1,)), ((), ())),
                        preferred_element_type=jnp.float32)
    dscr[...] = (sqb + (-2.0 * g)) + sqfscr[...]

    colid = lax.broadcasted_iota(jnp.int32, (ROWS, N), 1)
    lane16 = lax.broadcasted_iota(jnp.int32, (ROWS, KNN_K), 1)
    inf = jnp.float32(jnp.inf)

    def step(j, acc):
        dc = dscr[...]
        m = jnp.min(dc, axis=1, keepdims=True)
        am = jnp.min(jnp.where(dc == m, colid, N), axis=1, keepdims=True)
        dscr[...] = jnp.where(colid == am, inf, dc)
        keep = (j % 2 == 0) & (lane16 == (j // 2))
        return jnp.where(keep, am, acc)

    acc = lax.fori_loop(0, NUM_EXTRACT, step,
                        jnp.zeros((ROWS, KNN_K), jnp.int32))
    idx_ref[0] = acc


def _dilated_knn_idx(new_points):
    return pl.pallas_call(
        _topk_body,
        grid=(B, N // ROWS),
        in_specs=[
            pl.BlockSpec((1, ROWS, C), lambda b, r: (b, r, 0)),
            pl.BlockSpec((1, N, C), lambda b, r: (b, 0, 0)),
        ],
        out_specs=pl.BlockSpec((1, ROWS, KNN_K), lambda b, r: (b, r, 0)),
        out_shape=jax.ShapeDtypeStruct((B, N, KNN_K), jnp.int32),
        scratch_shapes=[pltpu.VMEM((ROWS, N), jnp.float32),
                        pltpu.VMEM((N, C), jnp.bfloat16),
                        pltpu.VMEM((1, N), jnp.float32)],
    )(new_points, new_points)


def _sqrt16(d2):
    bits = lax.bitcast_convert_type(d2, jnp.int32)
    y = lax.bitcast_convert_type(
        jnp.int32(0x5F3759DF) - (bits >> 1), jnp.float32)
    for _ in range(3):
        y = y * (1.5 - 0.5 * d2 * y * y)
    return jnp.where(d2 > 0.0, d2 * y, 0.0)


def _encode_body(xyz_ref, nn_ref, out_ref, tv, iv, ov):
    cid = lax.axis_index("c")
    sid = lax.axis_index("s")
    w = sid * 2 + cid
    pltpu.sync_copy(xyz_ref, tv)
    pltpu.sync_copy(nn_ref.at[pl.ds(w * (PTS_PER_W * KNN_K), PTS_PER_W * KNN_K)], iv)
    base = (w // (N // PTS_PER_W)) * N
    lanes = jnp.arange(16, dtype=jnp.int32)

    for sub in range(PTS_PER_W // CHUNK_PTS):
        def body(p2, carry):
            p_loc = sub * CHUNK_PTS + p2
            idx16 = iv[pl.ds(p_loc * KNN_K, KNN_K)]
            gi = (idx16 + base) * 3
            nx = plsc.load_gather(tv, [gi])
            ny = plsc.load_gather(tv, [gi + 1])
            nz = plsc.load_gather(tv, [gi + 2])
            pw = (w * PTS_PER_W + p_loc) * 3
            own_i = jnp.full((16,), pw, jnp.int32)
            ox = plsc.load_gather(tv, [own_i])
            oy = plsc.load_gather(tv, [own_i + 1])
            oz = plsc.load_gather(tv, [own_i + 2])
            rx = ox - nx
            ry = oy - ny
            rz = oz - nz
            dist = _sqrt16(rx * rx + ry * ry + rz * rz)
            offs = p2 * (KNN_K * OUT_CH) + lanes * OUT_CH
            plsc.store_scatter(ov, [offs], dist)
            plsc.store_scatter(ov, [offs + 1], rx)
            plsc.store_scatter(ov, [offs + 2], ry)
            plsc.store_scatter(ov, [offs + 3], rz)
            plsc.store_scatter(ov, [offs + 4], ox)
            plsc.store_scatter(ov, [offs + 5], oy)
            plsc.store_scatter(ov, [offs + 6], oz)
            plsc.store_scatter(ov, [offs + 7], nx)
            plsc.store_scatter(ov, [offs + 8], ny)
            plsc.store_scatter(ov, [offs + 9], nz)
            return carry

        lax.fori_loop(0, CHUNK_PTS, body, 0)
        pltpu.sync_copy(
            ov, out_ref.at[pl.ds(w * (PTS_PER_W * KNN_K * OUT_CH)
                                 + sub * CHUNK_F32, CHUNK_F32)])


def _relative_pos_encode(xyz_flat, nn_flat):
    mesh = plsc.VectorSubcoreMesh(core_axis_name="c", subcore_axis_name="s")
    run = functools.partial(
        pl.kernel,
        mesh=mesh,
        compiler_params=pltpu.CompilerParams(needs_layout_passes=False),
        out_type=jax.ShapeDtypeStruct((B * N * KNN_K * OUT_CH,), jnp.float32),
        scratch_types=[
            pltpu.VMEM((B * N * 3,), jnp.float32),
            pltpu.VMEM((PTS_PER_W * KNN_K,), jnp.int32),
            pltpu.VMEM((CHUNK_F32,), jnp.float32),
        ],
    )(_encode_body)
    return run(xyz_flat, nn_flat)


def kernel(new_xyz, new_points):
    nn_idx = _dilated_knn_idx(new_points)
    out = _relative_pos_encode(new_xyz.reshape(-1), nn_idx.reshape(-1))
    return out.reshape(B, N, KNN_K, OUT_CH)
